# baseline (speedup 1.0000x reference)
import functools

import jax
import jax.numpy as jnp
from jax import lax
from jax.experimental import pallas as pl
from jax.experimental.pallas import tpu as pltpu
from jax.experimental.pallas import tpu_sc as plsc

N = 10000
E = 2048
D = 128
DE = 16
DG = 8

NC = 2
NS = 16
NW = NC * NS
EPT = E // NW
NPAD = 10240
ROWS_PT = NPAD // NS

_sc_mesh = plsc.VectorSubcoreMesh(core_axis_name="c", subcore_axis_name="s")


@functools.partial(
    pl.kernel,
    out_type=(jax.ShapeDtypeStruct((E, D), jnp.float32),
              jax.ShapeDtypeStruct((E, D), jnp.float32)),
    mesh=_sc_mesh,
    scratch_types=[
        pltpu.VMEM((EPT,), jnp.int32),
        pltpu.VMEM((EPT,), jnp.int32),
        pltpu.VMEM((EPT, D), jnp.float32),
        pltpu.VMEM((EPT, D), jnp.float32),
        pltpu.SemaphoreType.DMA,
        pltpu.SemaphoreType.DMA,
    ],
)
def _sc_gather(nodes_hbm, send_hbm, recv_hbm, out_s, out_r,
               idx_s, idx_r, rows_s, rows_r, sem_s, sem_r):
    wid = lax.axis_index("c") * NS + lax.axis_index("s")
    base = wid * EPT
    pltpu.sync_copy(send_hbm.at[pl.ds(base, EPT)], idx_s)
    pltpu.sync_copy(recv_hbm.at[pl.ds(base, EPT)], idx_r)
    cp_s = pltpu.async_copy(nodes_hbm.at[idx_s], rows_s, sem_s)
    cp_r = pltpu.async_copy(nodes_hbm.at[idx_r], rows_r, sem_r)
    cp_s.wait()
    cp_r.wait()
    pltpu.sync_copy(rows_s, out_s.at[pl.ds(base, EPT)])
    pltpu.sync_copy(rows_r, out_r.at[pl.ds(base, EPT)])


HALF = NPAD // 2
ACC_ROWS = 5248
ZPT = ACC_ROWS // NS
OPT = HALF // NS
EPTC = E // NS


@functools.partial(
    pl.kernel,
    out_type=jax.ShapeDtypeStruct((NPAD, D), jnp.float32),
    mesh=_sc_mesh,
    scratch_types=[
        pltpu.VMEM((EPTC,), jnp.int32),
        pltpu.VMEM((EPTC,), jnp.int32),
        pltpu.VMEM((EPTC, D), jnp.float32),
        pltpu.VMEM((EPTC, D), jnp.float32),
        pltpu.VMEM_SHARED((ACC_ROWS, D), jnp.float32),
    ],
)
def _sc_scatter(zeros_hbm, ps_hbm, pr_hbm, send_hbm, recv_hbm, out_hbm,
                idx_s, idx_r, rows_s, rows_r, acc):
    c = lax.axis_index("c")
    s = lax.axis_index("s")
    ebase = s * EPTC
    lo = c * HALF
    pltpu.sync_copy(zeros_hbm.at[pl.ds(s * ZPT, ZPT)],
                    acc.at[pl.ds(s * ZPT, ZPT)])
    pltpu.sync_copy(send_hbm.at[pl.ds(ebase, EPTC)], idx_s)
    pltpu.sync_copy(recv_hbm.at[pl.ds(ebase, EPTC)], idx_r)
    pltpu.sync_copy(ps_hbm.at[pl.ds(ebase, EPTC)], rows_s)
    pltpu.sync_copy(pr_hbm.at[pl.ds(ebase, EPTC)], rows_r)
    for j in range(EPTC // 16):
        sl = pl.ds(j * 16, 16)
        for idx_ref in (idx_s, idx_r):
            v = idx_ref[sl] - lo
            inb = (v >= 0) & (v < HALF)
            idx_ref[sl] = jnp.where(inb, v, HALF)
    plsc.subcore_barrier()
    pltpu.sync_copy(rows_s, acc.at[idx_s], add=True)
    pltpu.sync_copy(rows_r, acc.at[idx_r], add=True)
    plsc.subcore_barrier()
    pltpu.sync_copy(acc.at[pl.ds(s * OPT, OPT)],
                    out_hbm.at[pl.ds(c * HALF + s * OPT, OPT)])


E_NBLK = 256

def _edge_body(g_ref, e_ref, s_ref, r_ref, w1_ref, b1_ref, w2_ref, b2_ref,
               ws_ref, wr_ref, enew_ref, ps_ref, pr_ref, h_ref):
    j = pl.program_id(0)

    @pl.when(j == 0)
    def _():
        ge = jnp.dot(g_ref[...], w1_ref[DE + 2 * D:, :],
                     preferred_element_type=jnp.float32) + b1_ref[...]
        h = jnp.dot(e_ref[...], w1_ref[:DE, :],
                    preferred_element_type=jnp.float32)
        h = h + jnp.dot(s_ref[...], w1_ref[DE:DE + D, :],
                        preferred_element_type=jnp.float32)
        h = h + jnp.dot(r_ref[...], w1_ref[DE + D:DE + 2 * D, :],
                        preferred_element_type=jnp.float32)
        h_ref[...] = jnp.maximum(h + ge, 0.0)

    e2 = jnp.maximum(
        jnp.dot(h_ref[...], w2_ref[...], preferred_element_type=jnp.float32)
        + b2_ref[...], 0.0)
    enew_ref[...] = e2
    pps = jnp.dot(e2, ws_ref[...], preferred_element_type=jnp.float32)
    ppr = jnp.dot(e2, wr_ref[...], preferred_element_type=jnp.float32)

    @pl.when(j == 0)
    def _():
        ps_ref[...] = pps
        pr_ref[...] = ppr

    @pl.when(j > 0)
    def _():
        ps_ref[...] += pps
        pr_ref[...] += ppr


def _edge_stage(globals_, edges, sent, recv, W_e1, b_e1, W_e2, b_e2,
                W_s, W_r):
    in_e = DE + 2 * D + DG
    full = lambda shape: pl.BlockSpec(shape, lambda j: (0, 0))
    return pl.pallas_call(
        _edge_body,
        grid=(E // E_NBLK,),
        in_specs=[
            full((1, DG)),
            full((E, DE)),
            full((E, D)),
            full((E, D)),
            full((in_e, E)),
            full((1, E)),
            pl.BlockSpec((E, E_NBLK), lambda j: (0, j)),
            pl.BlockSpec((1, E_NBLK), lambda j: (0, j)),
            pl.BlockSpec((E_NBLK, D), lambda j: (j, 0)),
            pl.BlockSpec((E_NBLK, D), lambda j: (j, 0)),
        ],
        out_specs=[
            pl.BlockSpec((E, E_NBLK), lambda j: (0, j)),
            pl.BlockSpec((E, D), lambda j: (0, 0)),
            pl.BlockSpec((E, D), lambda j: (0, 0)),
        ],
        out_shape=[
            jax.ShapeDtypeStruct((E, E), jnp.float32),
            jax.ShapeDtypeStruct((E, D), jnp.float32),
            jax.ShapeDtypeStruct((E, D), jnp.float32),
        ],
        scratch_shapes=[pltpu.VMEM((E, E), jnp.float32)],
    )(globals_, edges, sent, recv, W_e1, b_e1[None, :], W_e2, b_e2[None, :],
      W_s, W_r)


N_BLK = 1000

def _node_body(g_ref, x_ref, p_ref, wn_ref, b1_ref,
               w2_ref, b2_ref, out_ref):
    gb = jnp.dot(g_ref[...], wn_ref[D + 2 * E:, :],
                 preferred_element_type=jnp.float32) + b1_ref[...]
    h = jnp.dot(x_ref[...], wn_ref[:D, :], preferred_element_type=jnp.float32)
    h = jnp.maximum(h + p_ref[...] + gb, 0.0)
    out_ref[...] = jnp.maximum(
        jnp.dot(h, w2_ref[...], preferred_element_type=jnp.float32)
        + b2_ref[...], 0.0)


def _node_stage(globals_, nodes, parts, W_n1, b_n1, W_n2, b_n2):
    in_n = D + 2 * E + DG
    full = lambda shape: pl.BlockSpec(shape, lambda i: (0, 0))
    return pl.pallas_call(
        _node_body,
        grid=(N // N_BLK,),
        in_specs=[
            full((1, DG)),
            pl.BlockSpec((N_BLK, D), lambda i: (i, 0)),
            pl.BlockSpec((N_BLK, D), lambda i: (i, 0)),
            full((in_n, D)),
            full((1, D)),
            full((D, D)),
            full((1, D)),
        ],
        out_specs=pl.BlockSpec((N_BLK, D), lambda i: (i, 0)),
        out_shape=jax.ShapeDtypeStruct((N, D), jnp.float32),
    )(globals_, nodes, parts, W_n1, b_n1[None, :], W_n2, b_n2[None, :])


def kernel(nodes, edges, receivers, senders, globals_, n_node, n_edge,
           W_e1, b_e1, W_e2, b_e2, W_n1, b_n1, W_n2, b_n2):
    sent, recv = _sc_gather(nodes, senders, receivers)

    edges_new, Ps, Pr = _edge_stage(globals_, edges, sent, recv,
                                    W_e1, b_e1, W_e2, b_e2,
                                    W_n1[D:D + E], W_n1[D + E:D + 2 * E])

    zeros = jnp.zeros((ACC_ROWS, D), jnp.float32)
    part = _sc_scatter(zeros, Ps, Pr, senders, receivers)

    nodes_new = _node_stage(globals_, nodes, part, W_n1, b_n1, W_n2, b_n2)
    return (nodes_new, edges_new, receivers, senders, globals_, n_node, n_edge)


# speedup vs baseline: 1.0568x; 1.0568x over previous
import functools

import jax
import jax.numpy as jnp
from jax import lax
from jax.experimental import pallas as pl
from jax.experimental.pallas import tpu as pltpu
from jax.experimental.pallas import tpu_sc as plsc

N = 10000
E = 2048
D = 128
DE = 16
DG = 8

NC = 2
NS = 16
NW = NC * NS
EPT = E // NW
NPAD = 10240
ROWS_PT = NPAD // NS

_sc_mesh = plsc.VectorSubcoreMesh(core_axis_name="c", subcore_axis_name="s")


@functools.partial(
    pl.kernel,
    out_type=(jax.ShapeDtypeStruct((E, D), jnp.float32),
              jax.ShapeDtypeStruct((E, D), jnp.float32)),
    mesh=_sc_mesh,
    scratch_types=[
        pltpu.VMEM((EPT,), jnp.int32),
        pltpu.VMEM((EPT,), jnp.int32),
        pltpu.VMEM((EPT, D), jnp.float32),
        pltpu.VMEM((EPT, D), jnp.float32),
        pltpu.SemaphoreType.DMA,
        pltpu.SemaphoreType.DMA,
    ],
)
def _sc_gather(nodes_hbm, send_hbm, recv_hbm, out_s, out_r,
               idx_s, idx_r, rows_s, rows_r, sem_s, sem_r):
    wid = lax.axis_index("c") * NS + lax.axis_index("s")
    base = wid * EPT
    pltpu.sync_copy(send_hbm.at[pl.ds(base, EPT)], idx_s)
    pltpu.sync_copy(recv_hbm.at[pl.ds(base, EPT)], idx_r)
    cp_s = pltpu.async_copy(nodes_hbm.at[idx_s], rows_s, sem_s)
    cp_r = pltpu.async_copy(nodes_hbm.at[idx_r], rows_r, sem_r)
    cp_s.wait()
    cp_r.wait()
    pltpu.sync_copy(rows_s, out_s.at[pl.ds(base, EPT)])
    pltpu.sync_copy(rows_r, out_r.at[pl.ds(base, EPT)])


HALF = NPAD // 2
ACC_ROWS = 5248
ZPT = ACC_ROWS // NS
OPT = HALF // NS
EPTC = E // NS


@functools.partial(
    pl.kernel,
    out_type=jax.ShapeDtypeStruct((NPAD, D), jnp.float32),
    mesh=_sc_mesh,
    scratch_types=[
        pltpu.VMEM((EPTC,), jnp.int32),
        pltpu.VMEM((EPTC,), jnp.int32),
        pltpu.VMEM((EPTC, D), jnp.float32),
        pltpu.VMEM((EPTC, D), jnp.float32),
        pltpu.VMEM_SHARED((ACC_ROWS, D), jnp.float32),
    ],
)
def _sc_scatter(zeros_hbm, ps_hbm, pr_hbm, send_hbm, recv_hbm, out_hbm,
                idx_s, idx_r, rows_s, rows_r, acc):
    c = lax.axis_index("c")
    s = lax.axis_index("s")
    ebase = s * EPTC
    lo = c * HALF
    pltpu.sync_copy(zeros_hbm.at[pl.ds(s * ZPT, ZPT)],
                    acc.at[pl.ds(s * ZPT, ZPT)])
    pltpu.sync_copy(send_hbm.at[pl.ds(ebase, EPTC)], idx_s)
    pltpu.sync_copy(recv_hbm.at[pl.ds(ebase, EPTC)], idx_r)
    pltpu.sync_copy(ps_hbm.at[pl.ds(ebase, EPTC)], rows_s)
    pltpu.sync_copy(pr_hbm.at[pl.ds(ebase, EPTC)], rows_r)
    for j in range(EPTC // 16):
        sl = pl.ds(j * 16, 16)
        for idx_ref in (idx_s, idx_r):
            v = idx_ref[sl] - lo
            inb = (v >= 0) & (v < HALF)
            idx_ref[sl] = jnp.where(inb, v, HALF)
    plsc.subcore_barrier()
    pltpu.sync_copy(rows_s, acc.at[idx_s], add=True)
    pltpu.sync_copy(rows_r, acc.at[idx_r], add=True)
    plsc.subcore_barrier()
    pltpu.sync_copy(acc.at[pl.ds(s * OPT, OPT)],
                    out_hbm.at[pl.ds(c * HALF + s * OPT, OPT)])


E_BLK = 256

def _edge_body(g_ref, e_ref, s_ref, r_ref, w1_ref, b1_ref, w2_ref, b2_ref,
               wn_ref, enew_ref, ps_ref, pr_ref):
    ge = jnp.dot(g_ref[...], w1_ref[DE + 2 * D:, :],
                 preferred_element_type=jnp.float32) + b1_ref[...]
    h = jnp.dot(e_ref[...], w1_ref[:DE, :], preferred_element_type=jnp.float32)
    h = h + jnp.dot(s_ref[...], w1_ref[DE:DE + D, :],
                    preferred_element_type=jnp.float32)
    h = h + jnp.dot(r_ref[...], w1_ref[DE + D:DE + 2 * D, :],
                    preferred_element_type=jnp.float32)
    h = jnp.maximum(h + ge, 0.0)
    e2 = jnp.maximum(jnp.dot(h, w2_ref[...], preferred_element_type=jnp.float32)
                     + b2_ref[...], 0.0)
    enew_ref[...] = e2
    ps_ref[...] = jnp.dot(e2, wn_ref[D:D + E, :],
                          preferred_element_type=jnp.float32)
    pr_ref[...] = jnp.dot(e2, wn_ref[D + E:D + 2 * E, :],
                          preferred_element_type=jnp.float32)


def _edge_stage(globals_, edges, sent, recv, W_e1, b_e1, W_e2, b_e2, W_n1):
    in_e = DE + 2 * D + DG
    in_n = D + 2 * E + DG
    full = lambda shape: pl.BlockSpec(shape, lambda i: (0, 0))
    return pl.pallas_call(
        _edge_body,
        grid=(E // E_BLK,),
        in_specs=[
            full((1, DG)),
            pl.BlockSpec((E_BLK, DE), lambda i: (i, 0)),
            pl.BlockSpec((E_BLK, D), lambda i: (i, 0)),
            pl.BlockSpec((E_BLK, D), lambda i: (i, 0)),
            full((in_e, E)),
            full((1, E)),
            full((E, E)),
            full((1, E)),
            full((in_n, D)),
        ],
        out_specs=[
            pl.BlockSpec((E_BLK, E), lambda i: (i, 0)),
            pl.BlockSpec((E_BLK, D), lambda i: (i, 0)),
            pl.BlockSpec((E_BLK, D), lambda i: (i, 0)),
        ],
        out_shape=[
            jax.ShapeDtypeStruct((E, E), jnp.float32),
            jax.ShapeDtypeStruct((E, D), jnp.float32),
            jax.ShapeDtypeStruct((E, D), jnp.float32),
        ],
    )(globals_, edges, sent, recv, W_e1, b_e1[None, :], W_e2, b_e2[None, :],
      W_n1)


N_BLK = 1000

def _node_body(g_ref, x_ref, p_ref, wn_ref, b1_ref,
               w2_ref, b2_ref, out_ref):
    gb = jnp.dot(g_ref[...], wn_ref[D + 2 * E:, :],
                 preferred_element_type=jnp.float32) + b1_ref[...]
    h = jnp.dot(x_ref[...], wn_ref[:D, :], preferred_element_type=jnp.float32)
    h = jnp.maximum(h + p_ref[...] + gb, 0.0)
    out_ref[...] = jnp.maximum(
        jnp.dot(h, w2_ref[...], preferred_element_type=jnp.float32)
        + b2_ref[...], 0.0)


def _node_stage(globals_, nodes, parts, W_n1, b_n1, W_n2, b_n2):
    in_n = D + 2 * E + DG
    full = lambda shape: pl.BlockSpec(shape, lambda i: (0, 0))
    return pl.pallas_call(
        _node_body,
        grid=(N // N_BLK,),
        in_specs=[
            full((1, DG)),
            pl.BlockSpec((N_BLK, D), lambda i: (i, 0)),
            pl.BlockSpec((N_BLK, D), lambda i: (i, 0)),
            full((in_n, D)),
            full((1, D)),
            full((D, D)),
            full((1, D)),
        ],
        out_specs=pl.BlockSpec((N_BLK, D), lambda i: (i, 0)),
        out_shape=jax.ShapeDtypeStruct((N, D), jnp.float32),
    )(globals_, nodes, parts, W_n1, b_n1[None, :], W_n2, b_n2[None, :])


def kernel(nodes, edges, receivers, senders, globals_, n_node, n_edge,
           W_e1, b_e1, W_e2, b_e2, W_n1, b_n1, W_n2, b_n2):
    sent, recv = _sc_gather(nodes, senders, receivers)

    edges_new, Ps, Pr = _edge_stage(globals_, edges, sent, recv,
                                    W_e1, b_e1, W_e2, b_e2, W_n1)

    zeros = jnp.zeros((ACC_ROWS, D), jnp.float32)
    part = _sc_scatter(zeros, Ps, Pr, senders, receivers)

    nodes_new = _node_stage(globals_, nodes, part, W_n1, b_n1, W_n2, b_n2)
    return (nodes_new, edges_new, receivers, senders, globals_, n_node, n_edge)


# speedup vs baseline: 1.0647x; 1.0075x over previous
import functools

import jax
import jax.numpy as jnp
from jax import lax
from jax.experimental import pallas as pl
from jax.experimental.pallas import tpu as pltpu
from jax.experimental.pallas import tpu_sc as plsc

N = 10000
E = 2048
D = 128
DE = 16
DG = 8

NC = 2
NS = 16
NW = NC * NS
EPT = E // NW
NPAD = 10240
ROWS_PT = NPAD // NS

_sc_mesh = plsc.VectorSubcoreMesh(core_axis_name="c", subcore_axis_name="s")


@functools.partial(
    pl.kernel,
    out_type=(jax.ShapeDtypeStruct((E, D), jnp.float32),
              jax.ShapeDtypeStruct((E, D), jnp.float32)),
    mesh=_sc_mesh,
    scratch_types=[
        pltpu.VMEM((EPT,), jnp.int32),
        pltpu.VMEM((EPT,), jnp.int32),
        pltpu.VMEM((EPT, D), jnp.float32),
        pltpu.VMEM((EPT, D), jnp.float32),
        pltpu.SemaphoreType.DMA,
        pltpu.SemaphoreType.DMA,
    ],
)
def _sc_gather(nodes_hbm, send_hbm, recv_hbm, out_s, out_r,
               idx_s, idx_r, rows_s, rows_r, sem_s, sem_r):
    wid = lax.axis_index("c") * NS + lax.axis_index("s")
    base = wid * EPT
    pltpu.sync_copy(send_hbm.at[pl.ds(base, EPT)], idx_s)
    pltpu.sync_copy(recv_hbm.at[pl.ds(base, EPT)], idx_r)
    cp_s = pltpu.async_copy(nodes_hbm.at[idx_s], rows_s, sem_s)
    cp_r = pltpu.async_copy(nodes_hbm.at[idx_r], rows_r, sem_r)
    cp_s.wait()
    cp_r.wait()
    pltpu.sync_copy(rows_s, out_s.at[pl.ds(base, EPT)])
    pltpu.sync_copy(rows_r, out_r.at[pl.ds(base, EPT)])


HALF = NPAD // 2
ACC_ROWS = 5248
ZPT = ACC_ROWS // NS
OPT = HALF // NS
EPTC = E // NS


@functools.partial(
    pl.kernel,
    out_type=jax.ShapeDtypeStruct((NPAD, D), jnp.float32),
    mesh=_sc_mesh,
    scratch_types=[
        pltpu.VMEM((EPTC,), jnp.int32),
        pltpu.VMEM((EPTC,), jnp.int32),
        pltpu.VMEM((EPTC, D), jnp.float32),
        pltpu.VMEM((EPTC, D), jnp.float32),
        pltpu.VMEM_SHARED((ACC_ROWS, D), jnp.float32),
    ],
)
def _sc_scatter(zeros_hbm, ps_hbm, pr_hbm, send_hbm, recv_hbm, out_hbm,
                idx_s, idx_r, rows_s, rows_r, acc):
    c = lax.axis_index("c")
    s = lax.axis_index("s")
    ebase = s * EPTC
    lo = c * HALF
    pltpu.sync_copy(zeros_hbm.at[pl.ds(s * ZPT, ZPT)],
                    acc.at[pl.ds(s * ZPT, ZPT)])
    pltpu.sync_copy(send_hbm.at[pl.ds(ebase, EPTC)], idx_s)
    pltpu.sync_copy(recv_hbm.at[pl.ds(ebase, EPTC)], idx_r)
    pltpu.sync_copy(ps_hbm.at[pl.ds(ebase, EPTC)], rows_s)
    pltpu.sync_copy(pr_hbm.at[pl.ds(ebase, EPTC)], rows_r)
    for j in range(EPTC // 16):
        sl = pl.ds(j * 16, 16)
        for idx_ref in (idx_s, idx_r):
            v = idx_ref[sl] - lo
            inb = (v >= 0) & (v < HALF)
            idx_ref[sl] = jnp.where(inb, v, HALF)
    plsc.subcore_barrier()
    pltpu.sync_copy(rows_s, acc.at[idx_s], add=True)
    pltpu.sync_copy(rows_r, acc.at[idx_r], add=True)
    plsc.subcore_barrier()
    pltpu.sync_copy(acc.at[pl.ds(s * OPT, OPT)],
                    out_hbm.at[pl.ds(c * HALF + s * OPT, OPT)])


E_BLK = 256

def _edge_body(g_ref, e_ref, s_ref, r_ref, w1_ref, b1_ref, w2_ref, b2_ref,
               wn_ref, enew_ref, ps_ref, pr_ref):
    ge = jnp.dot(g_ref[...], w1_ref[DE + 2 * D:, :],
                 preferred_element_type=jnp.float32) + b1_ref[...]
    h = jnp.dot(e_ref[...], w1_ref[:DE, :], preferred_element_type=jnp.float32)
    h = h + jnp.dot(s_ref[...], w1_ref[DE:DE + D, :],
                    preferred_element_type=jnp.float32)
    h = h + jnp.dot(r_ref[...], w1_ref[DE + D:DE + 2 * D, :],
                    preferred_element_type=jnp.float32)
    h = jnp.maximum(h + ge, 0.0)
    e2 = jnp.maximum(jnp.dot(h, w2_ref[...], preferred_element_type=jnp.float32)
                     + b2_ref[...], 0.0)
    enew_ref[...] = e2
    ps_ref[...] = jnp.dot(e2, wn_ref[D:D + E, :],
                          preferred_element_type=jnp.float32)
    pr_ref[...] = jnp.dot(e2, wn_ref[D + E:D + 2 * E, :],
                          preferred_element_type=jnp.float32)


def _edge_stage(globals_, edges, sent, recv, W_e1, b_e1, W_e2, b_e2, W_n1):
    in_e = DE + 2 * D + DG
    in_n = D + 2 * E + DG
    full = lambda shape: pl.BlockSpec(shape, lambda i: (0, 0))
    return pl.pallas_call(
        _edge_body,
        grid=(E // E_BLK,),
        in_specs=[
            full((1, DG)),
            pl.BlockSpec((E_BLK, DE), lambda i: (i, 0)),
            pl.BlockSpec((E_BLK, D), lambda i: (i, 0)),
            pl.BlockSpec((E_BLK, D), lambda i: (i, 0)),
            full((in_e, E)),
            full((1, E)),
            full((E, E)),
            full((1, E)),
            full((in_n, D)),
        ],
        out_specs=[
            pl.BlockSpec((E_BLK, E), lambda i: (i, 0)),
            pl.BlockSpec((E_BLK, D), lambda i: (i, 0)),
            pl.BlockSpec((E_BLK, D), lambda i: (i, 0)),
        ],
        out_shape=[
            jax.ShapeDtypeStruct((E, E), jnp.float32),
            jax.ShapeDtypeStruct((E, D), jnp.float32),
            jax.ShapeDtypeStruct((E, D), jnp.float32),
        ],
    )(globals_, edges, sent, recv, W_e1, b_e1[None, :], W_e2, b_e2[None, :],
      W_n1)


N_BLK = 1000

def _node_body(g_ref, x_ref, p_ref, wn_ref, wg_ref, b1_ref,
               w2_ref, b2_ref, out_ref):
    gb = jnp.dot(g_ref[...], wg_ref[...],
                 preferred_element_type=jnp.float32) + b1_ref[...]
    h = jnp.dot(x_ref[...], wn_ref[...], preferred_element_type=jnp.float32)
    h = jnp.maximum(h + p_ref[...] + gb, 0.0)
    out_ref[...] = jnp.maximum(
        jnp.dot(h, w2_ref[...], preferred_element_type=jnp.float32)
        + b2_ref[...], 0.0)


def _node_stage(globals_, nodes, parts, W_n1, b_n1, W_n2, b_n2):
    full = lambda shape: pl.BlockSpec(shape, lambda i: (0, 0))
    return pl.pallas_call(
        _node_body,
        grid=(N // N_BLK,),
        in_specs=[
            full((1, DG)),
            pl.BlockSpec((N_BLK, D), lambda i: (i, 0)),
            pl.BlockSpec((N_BLK, D), lambda i: (i, 0)),
            full((D, D)),
            full((DG, D)),
            full((1, D)),
            full((D, D)),
            full((1, D)),
        ],
        out_specs=pl.BlockSpec((N_BLK, D), lambda i: (i, 0)),
        out_shape=jax.ShapeDtypeStruct((N, D), jnp.float32),
    )(globals_, nodes, parts, W_n1[:D], W_n1[D + 2 * E:], b_n1[None, :],
      W_n2, b_n2[None, :])


def kernel(nodes, edges, receivers, senders, globals_, n_node, n_edge,
           W_e1, b_e1, W_e2, b_e2, W_n1, b_n1, W_n2, b_n2):
    sent, recv = _sc_gather(nodes, senders, receivers)

    edges_new, Ps, Pr = _edge_stage(globals_, edges, sent, recv,
                                    W_e1, b_e1, W_e2, b_e2, W_n1)

    zeros = jnp.zeros((ACC_ROWS, D), jnp.float32)
    part = _sc_scatter(zeros, Ps, Pr, senders, receivers)

    nodes_new = _node_stage(globals_, nodes, part, W_n1, b_n1, W_n2, b_n2)
    return (nodes_new, edges_new, receivers, senders, globals_, n_node, n_edge)
